# baseline (device time: 47958 ns/iter reference)
import jax
import jax.numpy as jnp
from jax import lax
from jax.experimental import pallas as pl
from jax.experimental.pallas import tpu as pltpu

N_DEV = 4
N_TOK = 512
D_IN = 256
D_OUT = 512
N_EXP = 8


def kernel(x, router_W, route_idx, expert_W):
    def body(x_ref, rw_ref, idx_ref, ew_ref, out_ref, comm_ref, send_sems, recv_sems):
        my = lax.axis_index("i")
        left = lax.rem(my + N_DEV - 1, N_DEV)
        right = lax.rem(my + 1, N_DEV)

        barrier_sem = pltpu.get_barrier_semaphore()
        for nbr in (left, right):
            pl.semaphore_signal(
                barrier_sem, inc=1,
                device_id=(nbr,), device_id_type=pl.DeviceIdType.MESH,
            )
        pl.semaphore_wait(barrier_sem, 2)

        xv = x_ref[:, :]
        scores = jnp.dot(xv, rw_ref[:, :], preferred_element_type=jnp.float32)
        s_max = jnp.max(scores, axis=-1, keepdims=True)
        p = jnp.exp(scores - s_max)
        p = p / jnp.sum(p, axis=-1, keepdims=True)

        iota = lax.broadcasted_iota(jnp.int32, (N_TOK, N_EXP), 1)
        ind = (idx_ref[:, 0:1] == iota) | (idx_ref[:, 1:2] == iota)
        w = jnp.where(ind, p, 0.0)
        w = w / jnp.sum(w, axis=-1, keepdims=True)

        e0 = 2 * my
        w0 = jnp.sum(jnp.where(iota == e0, w, 0.0), axis=1, keepdims=True)
        w1 = jnp.sum(jnp.where(iota == e0 + 1, w, 0.0), axis=1, keepdims=True)

        partial = jnp.dot(xv * w0, ew_ref[0], preferred_element_type=jnp.float32)
        partial = partial + jnp.dot(
            xv * w1, ew_ref[1], preferred_element_type=jnp.float32
        )
        comm_ref[0] = partial

        for h in range(N_DEV - 1):
            rdma = pltpu.make_async_remote_copy(
                src_ref=comm_ref.at[h],
                dst_ref=comm_ref.at[h + 1],
                send_sem=send_sems.at[h],
                recv_sem=recv_sems.at[h],
                device_id=(right,),
                device_id_type=pl.DeviceIdType.MESH,
            )
            rdma.start()
            rdma.wait()

        out_ref[:, :] = (
            comm_ref[0] + comm_ref[1] + comm_ref[2] + comm_ref[3]
        )

    return pl.pallas_call(
        body,
        out_shape=jax.ShapeDtypeStruct((N_TOK, D_OUT), jnp.float32),
        in_specs=[
            pl.BlockSpec(memory_space=pltpu.VMEM),
            pl.BlockSpec(memory_space=pltpu.VMEM),
            pl.BlockSpec(memory_space=pltpu.VMEM),
            pl.BlockSpec(memory_space=pltpu.VMEM),
        ],
        out_specs=pl.BlockSpec(memory_space=pltpu.VMEM),
        scratch_shapes=[
            pltpu.VMEM((N_DEV, N_TOK, D_OUT), jnp.float32),
            pltpu.SemaphoreType.DMA((N_DEV - 1,)),
            pltpu.SemaphoreType.DMA((N_DEV - 1,)),
        ],
        compiler_params=pltpu.CompilerParams(collective_id=0),
    )(x, router_W, route_idx, expert_W)


# device time: 32241 ns/iter; 1.4875x vs baseline; 1.4875x over previous
import jax
import jax.numpy as jnp
from jax import lax
from jax.experimental import pallas as pl
from jax.experimental.pallas import tpu as pltpu

N_DEV = 4
N_TOK = 512
D_IN = 256
D_OUT = 512
N_EXP = 8

HALF = N_TOK // 2
QTR = N_TOK // 4


def kernel(x, router_W, route_idx, expert_W):
    def body(x_ref, rw_ref, idx_ref, ew_ref, out_ref, recv_h_ref, recv_q_ref,
             send_sems, recv_sems):
        my = lax.axis_index("i")
        partner1 = my ^ 1
        partner2 = my ^ 3

        barrier_sem = pltpu.get_barrier_semaphore()
        for nbr in (partner1, partner2):
            pl.semaphore_signal(
                barrier_sem, inc=1,
                device_id=(nbr,), device_id_type=pl.DeviceIdType.MESH,
            )
        pl.semaphore_wait(barrier_sem, 2)

        xv = x_ref[:, :]
        scores = jnp.dot(xv, rw_ref[:, :], preferred_element_type=jnp.float32)
        s_max = jnp.max(scores, axis=-1, keepdims=True)
        p = jnp.exp(scores - s_max)
        p = p / jnp.sum(p, axis=-1, keepdims=True)

        iota = lax.broadcasted_iota(jnp.int32, (N_TOK, N_EXP), 1)
        ind = (idx_ref[:, 0:1] == iota) | (idx_ref[:, 1:2] == iota)
        w = jnp.where(ind, p, 0.0)
        w = w / jnp.sum(w, axis=-1, keepdims=True)

        e0 = 2 * my
        w0 = jnp.sum(jnp.where(iota == e0, w, 0.0), axis=1, keepdims=True)
        w1 = jnp.sum(jnp.where(iota == e0 + 1, w, 0.0), axis=1, keepdims=True)

        partial = jnp.dot(xv * w0, ew_ref[0], preferred_element_type=jnp.float32)
        partial = partial + jnp.dot(
            xv * w1, ew_ref[1], preferred_element_type=jnp.float32
        )
        out_ref[:, :] = partial

        b1 = jnp.where((my == 1) | (my == 2), 1, 0)
        b2 = jnp.where(my >= 2, 1, 0)
        h0 = b1 * HALF
        hs = (1 - b1) * HALF
        q0 = h0 + b2 * QTR
        qs = h0 + (1 - b2) * QTR

        s1 = pltpu.make_async_remote_copy(
            src_ref=out_ref.at[pl.ds(hs, HALF), :],
            dst_ref=recv_h_ref,
            send_sem=send_sems.at[0],
            recv_sem=recv_sems.at[0],
            device_id=(partner1,),
            device_id_type=pl.DeviceIdType.MESH,
        )
        s1.start()
        s1.wait()
        out_ref[pl.ds(h0, HALF), :] = (
            out_ref[pl.ds(h0, HALF), :] + recv_h_ref[:, :]
        )

        s2 = pltpu.make_async_remote_copy(
            src_ref=out_ref.at[pl.ds(qs, QTR), :],
            dst_ref=recv_q_ref,
            send_sem=send_sems.at[1],
            recv_sem=recv_sems.at[1],
            device_id=(partner2,),
            device_id_type=pl.DeviceIdType.MESH,
        )
        s2.start()
        s2.wait()
        out_ref[pl.ds(q0, QTR), :] = (
            out_ref[pl.ds(q0, QTR), :] + recv_q_ref[:, :]
        )

        s3 = pltpu.make_async_remote_copy(
            src_ref=out_ref.at[pl.ds(q0, QTR), :],
            dst_ref=out_ref.at[pl.ds(q0, QTR), :],
            send_sem=send_sems.at[2],
            recv_sem=recv_sems.at[2],
            device_id=(partner2,),
            device_id_type=pl.DeviceIdType.MESH,
        )
        s3.start()
        s3.wait()

        s4 = pltpu.make_async_remote_copy(
            src_ref=out_ref.at[pl.ds(h0, HALF), :],
            dst_ref=out_ref.at[pl.ds(h0, HALF), :],
            send_sem=send_sems.at[3],
            recv_sem=recv_sems.at[3],
            device_id=(partner1,),
            device_id_type=pl.DeviceIdType.MESH,
        )
        s4.start()
        s4.wait()

    return pl.pallas_call(
        body,
        out_shape=jax.ShapeDtypeStruct((N_TOK, D_OUT), jnp.float32),
        in_specs=[
            pl.BlockSpec(memory_space=pltpu.VMEM),
            pl.BlockSpec(memory_space=pltpu.VMEM),
            pl.BlockSpec(memory_space=pltpu.VMEM),
            pl.BlockSpec(memory_space=pltpu.VMEM),
        ],
        out_specs=pl.BlockSpec(memory_space=pltpu.VMEM),
        scratch_shapes=[
            pltpu.VMEM((HALF, D_OUT), jnp.float32),
            pltpu.VMEM((QTR, D_OUT), jnp.float32),
            pltpu.SemaphoreType.DMA((4,)),
            pltpu.SemaphoreType.DMA((4,)),
        ],
        compiler_params=pltpu.CompilerParams(collective_id=0),
    )(x, router_W, route_idx, expert_W)


# device time: 21861 ns/iter; 2.1938x vs baseline; 1.4748x over previous
import jax
import jax.numpy as jnp
from jax import lax
from jax.experimental import pallas as pl
from jax.experimental.pallas import tpu as pltpu

N_DEV = 4
N_TOK = 512
D_IN = 256
D_OUT = 512
N_EXP = 8

N_CHUNK = 4
CHUNK = N_TOK // N_CHUNK
ORDER = (0, 2, 1, 3)


def kernel(x, router_W, route_idx, expert_W):
    def body(x_ref, rw_ref, idx_ref, ew_ref, out_ref, recv_a_ref, recv_b_ref,
             send_sems, recv_sems):
        my = lax.axis_index("i")
        partner1 = my ^ 1
        partner2 = my ^ 3

        barrier_sem = pltpu.get_barrier_semaphore()
        for nbr in (partner1, partner2):
            pl.semaphore_signal(
                barrier_sem, inc=1,
                device_id=(nbr,), device_id_type=pl.DeviceIdType.MESH,
            )
        pl.semaphore_wait(barrier_sem, 2)

        xv = x_ref[:, :]
        scores = jnp.dot(xv, rw_ref[:, :], preferred_element_type=jnp.float32)
        s_max = jnp.max(scores, axis=-1, keepdims=True)
        p = jnp.exp(scores - s_max)
        p = p / jnp.sum(p, axis=-1, keepdims=True)

        iota = lax.broadcasted_iota(jnp.int32, (N_TOK, N_EXP), 1)
        ind = (idx_ref[:, 0:1] == iota) | (idx_ref[:, 1:2] == iota)
        w = jnp.where(ind, p, 0.0)
        w = w / jnp.sum(w, axis=-1, keepdims=True)

        e0 = 2 * my
        w0 = jnp.sum(jnp.where(iota == e0, w, 0.0), axis=1, keepdims=True)
        w1 = jnp.sum(jnp.where(iota == e0 + 1, w, 0.0), axis=1, keepdims=True)
        a0 = xv * w0
        a1 = xv * w1

        ew0 = ew_ref[0]
        ew1 = ew_ref[1]

        rdma_a = {}
        for c in ORDER:
            r = c * CHUNK
            pa = partner1 if c < 2 else partner2
            chunk = jnp.dot(
                a0[r:r + CHUNK, :], ew0, preferred_element_type=jnp.float32
            ) + jnp.dot(
                a1[r:r + CHUNK, :], ew1, preferred_element_type=jnp.float32
            )
            out_ref[r:r + CHUNK, :] = chunk
            rd = pltpu.make_async_remote_copy(
                src_ref=out_ref.at[r:r + CHUNK, :],
                dst_ref=recv_a_ref.at[c],
                send_sem=send_sems.at[c],
                recv_sem=recv_sems.at[c],
                device_id=(pa,),
                device_id_type=pl.DeviceIdType.MESH,
            )
            rd.start()
            rdma_a[c] = rd

        rdma_b = {}
        for c in ORDER:
            rdma_a[c].wait()
            r = c * CHUNK
            out_ref[r:r + CHUNK, :] = out_ref[r:r + CHUNK, :] + recv_a_ref[c]
            pb = partner2 if c < 2 else partner1
            rd = pltpu.make_async_remote_copy(
                src_ref=out_ref.at[r:r + CHUNK, :],
                dst_ref=recv_b_ref.at[c],
                send_sem=send_sems.at[N_CHUNK + c],
                recv_sem=recv_sems.at[N_CHUNK + c],
                device_id=(pb,),
                device_id_type=pl.DeviceIdType.MESH,
            )
            rd.start()
            rdma_b[c] = rd

        for c in ORDER:
            rdma_b[c].wait()
            r = c * CHUNK
            out_ref[r:r + CHUNK, :] = out_ref[r:r + CHUNK, :] + recv_b_ref[c]

    return pl.pallas_call(
        body,
        out_shape=jax.ShapeDtypeStruct((N_TOK, D_OUT), jnp.float32),
        in_specs=[
            pl.BlockSpec(memory_space=pltpu.VMEM),
            pl.BlockSpec(memory_space=pltpu.VMEM),
            pl.BlockSpec(memory_space=pltpu.VMEM),
            pl.BlockSpec(memory_space=pltpu.VMEM),
        ],
        out_specs=pl.BlockSpec(memory_space=pltpu.VMEM),
        scratch_shapes=[
            pltpu.VMEM((N_CHUNK, CHUNK, D_OUT), jnp.float32),
            pltpu.VMEM((N_CHUNK, CHUNK, D_OUT), jnp.float32),
            pltpu.SemaphoreType.DMA((2 * N_CHUNK,)),
            pltpu.SemaphoreType.DMA((2 * N_CHUNK,)),
        ],
        compiler_params=pltpu.CompilerParams(collective_id=0),
    )(x, router_W, route_idx, expert_W)


# device time: 16179 ns/iter; 2.9642x vs baseline; 1.3512x over previous
import jax
import jax.numpy as jnp
from jax import lax
from jax.experimental import pallas as pl
from jax.experimental.pallas import tpu as pltpu

N_DEV = 4
N_TOK = 512
D_IN = 256
D_OUT = 512
N_EXP = 8

N_CHUNK = 4
CHUNK = N_TOK // N_CHUNK
ORDER = (0, 2, 1, 3)


def kernel(x, router_W, route_idx, expert_W):
    def body(x_ref, rw_ref, idx_ref, ew_ref, out_ref, send_a_ref, recv_a_ref,
             send_b_ref, recv_b_ref, send_sems, recv_sems):
        my = lax.axis_index("i")
        partner1 = my ^ 1
        partner2 = my ^ 3

        barrier_sem = pltpu.get_barrier_semaphore()
        for nbr in (partner1, partner2):
            pl.semaphore_signal(
                barrier_sem, inc=1,
                device_id=(nbr,), device_id_type=pl.DeviceIdType.MESH,
            )
        pl.semaphore_wait(barrier_sem, 2)

        xv = x_ref[:, :]
        scores = jnp.dot(xv, rw_ref[:, :], preferred_element_type=jnp.float32)
        s_max = jnp.max(scores, axis=-1, keepdims=True)
        p = jnp.exp(scores - s_max)
        p = p / jnp.sum(p, axis=-1, keepdims=True)

        iota = lax.broadcasted_iota(jnp.int32, (N_TOK, N_EXP), 1)
        ind = (idx_ref[:, 0:1] == iota) | (idx_ref[:, 1:2] == iota)
        w = jnp.where(ind, p, 0.0)
        w = w / jnp.sum(w, axis=-1, keepdims=True)

        e0 = 2 * my
        w0 = jnp.sum(jnp.where(iota == e0, w, 0.0), axis=1, keepdims=True)
        w1 = jnp.sum(jnp.where(iota == e0 + 1, w, 0.0), axis=1, keepdims=True)
        a0 = xv * w0
        a1 = xv * w1

        ew0 = ew_ref[0]
        ew1 = ew_ref[1]

        rdma_a = {}
        for c in ORDER:
            r = c * CHUNK
            pa = partner1 if c < 2 else partner2
            chunk = jnp.dot(
                a0[r:r + CHUNK, :], ew0, preferred_element_type=jnp.float32
            ) + jnp.dot(
                a1[r:r + CHUNK, :], ew1, preferred_element_type=jnp.float32
            )
            out_ref[r:r + CHUNK, :] = chunk
            send_a_ref[c] = chunk.astype(jnp.bfloat16)
            rd = pltpu.make_async_remote_copy(
                src_ref=send_a_ref.at[c],
                dst_ref=recv_a_ref.at[c],
                send_sem=send_sems.at[c],
                recv_sem=recv_sems.at[c],
                device_id=(pa,),
                device_id_type=pl.DeviceIdType.MESH,
            )
            rd.start()
            rdma_a[c] = rd

        rdma_b = {}
        for c in ORDER:
            rdma_a[c].wait()
            r = c * CHUNK
            pair_sum = out_ref[r:r + CHUNK, :] + recv_a_ref[c].astype(jnp.float32)
            out_ref[r:r + CHUNK, :] = pair_sum
            send_b_ref[c] = pair_sum.astype(jnp.bfloat16)
            pb = partner2 if c < 2 else partner1
            rd = pltpu.make_async_remote_copy(
                src_ref=send_b_ref.at[c],
                dst_ref=recv_b_ref.at[c],
                send_sem=send_sems.at[N_CHUNK + c],
                recv_sem=recv_sems.at[N_CHUNK + c],
                device_id=(pb,),
                device_id_type=pl.DeviceIdType.MESH,
            )
            rd.start()
            rdma_b[c] = rd

        for c in ORDER:
            rdma_b[c].wait()
            r = c * CHUNK
            out_ref[r:r + CHUNK, :] = (
                out_ref[r:r + CHUNK, :] + recv_b_ref[c].astype(jnp.float32)
            )

    return pl.pallas_call(
        body,
        out_shape=jax.ShapeDtypeStruct((N_TOK, D_OUT), jnp.float32),
        in_specs=[
            pl.BlockSpec(memory_space=pltpu.VMEM),
            pl.BlockSpec(memory_space=pltpu.VMEM),
            pl.BlockSpec(memory_space=pltpu.VMEM),
            pl.BlockSpec(memory_space=pltpu.VMEM),
        ],
        out_specs=pl.BlockSpec(memory_space=pltpu.VMEM),
        scratch_shapes=[
            pltpu.VMEM((N_CHUNK, CHUNK, D_OUT), jnp.bfloat16),
            pltpu.VMEM((N_CHUNK, CHUNK, D_OUT), jnp.bfloat16),
            pltpu.VMEM((N_CHUNK, CHUNK, D_OUT), jnp.bfloat16),
            pltpu.VMEM((N_CHUNK, CHUNK, D_OUT), jnp.bfloat16),
            pltpu.SemaphoreType.DMA((2 * N_CHUNK,)),
            pltpu.SemaphoreType.DMA((2 * N_CHUNK,)),
        ],
        compiler_params=pltpu.CompilerParams(collective_id=0),
    )(x, router_W, route_idx, expert_W)


# device time: 15620 ns/iter; 3.0703x vs baseline; 1.0358x over previous
import jax
import jax.numpy as jnp
from jax import lax
from jax.experimental import pallas as pl
from jax.experimental.pallas import tpu as pltpu

N_DEV = 4
N_TOK = 512
D_IN = 256
D_OUT = 512
N_EXP = 8

N_CHUNK = 4
CHUNK = N_TOK // N_CHUNK
ORDER = (0, 2, 1, 3)


def kernel(x, router_W, route_idx, expert_W):
    def body(x_ref, rw_ref, idx_ref, ew_ref, out_ref, send_a_ref, recv_a_ref,
             send_b_ref, recv_b_ref, send_sems, recv_sems):
        my = lax.axis_index("i")
        partner1 = my ^ 1
        partner2 = my ^ 3

        barrier_sem = pltpu.get_barrier_semaphore()
        for nbr in (partner1, partner2):
            pl.semaphore_signal(
                barrier_sem, inc=1,
                device_id=(nbr,), device_id_type=pl.DeviceIdType.MESH,
            )

        xv = x_ref[:, :]
        scores = jnp.dot(xv, rw_ref[:, :], preferred_element_type=jnp.float32)
        s_max = jnp.max(scores, axis=-1, keepdims=True)
        p = jnp.exp(scores - s_max)
        p = p / jnp.sum(p, axis=-1, keepdims=True)

        iota = lax.broadcasted_iota(jnp.int32, (N_TOK, N_EXP), 1)
        ind = (idx_ref[:, 0:1] == iota) | (idx_ref[:, 1:2] == iota)
        w = jnp.where(ind, p, 0.0)
        w = w / jnp.sum(w, axis=-1, keepdims=True)

        e0 = 2 * my
        w0 = jnp.sum(jnp.where(iota == e0, w, 0.0), axis=1, keepdims=True)
        w1 = jnp.sum(jnp.where(iota == e0 + 1, w, 0.0), axis=1, keepdims=True)
        a0 = (xv * w0).astype(jnp.bfloat16)
        a1 = (xv * w1).astype(jnp.bfloat16)

        ew0 = ew_ref[0].astype(jnp.bfloat16)
        ew1 = ew_ref[1].astype(jnp.bfloat16)

        pl.semaphore_wait(barrier_sem, 2)

        rdma_a = {}
        for c in ORDER:
            r = c * CHUNK
            pa = partner1 if c < 2 else partner2
            chunk = jnp.dot(
                a0[r:r + CHUNK, :], ew0, preferred_element_type=jnp.float32
            ) + jnp.dot(
                a1[r:r + CHUNK, :], ew1, preferred_element_type=jnp.float32
            )
            out_ref[r:r + CHUNK, :] = chunk
            send_a_ref[c] = chunk.astype(jnp.bfloat16)
            rd = pltpu.make_async_remote_copy(
                src_ref=send_a_ref.at[c],
                dst_ref=recv_a_ref.at[c],
                send_sem=send_sems.at[c],
                recv_sem=recv_sems.at[c],
                device_id=(pa,),
                device_id_type=pl.DeviceIdType.MESH,
            )
            rd.start()
            rdma_a[c] = rd

        rdma_b = {}
        for c in ORDER:
            rdma_a[c].wait()
            r = c * CHUNK
            pair_sum = out_ref[r:r + CHUNK, :] + recv_a_ref[c].astype(jnp.float32)
            out_ref[r:r + CHUNK, :] = pair_sum
            send_b_ref[c] = pair_sum.astype(jnp.bfloat16)
            pb = partner2 if c < 2 else partner1
            rd = pltpu.make_async_remote_copy(
                src_ref=send_b_ref.at[c],
                dst_ref=recv_b_ref.at[c],
                send_sem=send_sems.at[N_CHUNK + c],
                recv_sem=recv_sems.at[N_CHUNK + c],
                device_id=(pb,),
                device_id_type=pl.DeviceIdType.MESH,
            )
            rd.start()
            rdma_b[c] = rd

        for c in ORDER:
            rdma_b[c].wait()
            r = c * CHUNK
            out_ref[r:r + CHUNK, :] = (
                out_ref[r:r + CHUNK, :] + recv_b_ref[c].astype(jnp.float32)
            )

    return pl.pallas_call(
        body,
        out_shape=jax.ShapeDtypeStruct((N_TOK, D_OUT), jnp.float32),
        in_specs=[
            pl.BlockSpec(memory_space=pltpu.VMEM),
            pl.BlockSpec(memory_space=pltpu.VMEM),
            pl.BlockSpec(memory_space=pltpu.VMEM),
            pl.BlockSpec(memory_space=pltpu.VMEM),
        ],
        out_specs=pl.BlockSpec(memory_space=pltpu.VMEM),
        scratch_shapes=[
            pltpu.VMEM((N_CHUNK, CHUNK, D_OUT), jnp.bfloat16),
            pltpu.VMEM((N_CHUNK, CHUNK, D_OUT), jnp.bfloat16),
            pltpu.VMEM((N_CHUNK, CHUNK, D_OUT), jnp.bfloat16),
            pltpu.VMEM((N_CHUNK, CHUNK, D_OUT), jnp.bfloat16),
            pltpu.SemaphoreType.DMA((2 * N_CHUNK,)),
            pltpu.SemaphoreType.DMA((2 * N_CHUNK,)),
        ],
        compiler_params=pltpu.CompilerParams(collective_id=0),
    )(x, router_W, route_idx, expert_W)
